# baseline (speedup 1.0000x reference)
import functools

import jax
import jax.numpy as jnp
from jax import lax
from jax.experimental import pallas as pl
from jax.experimental.pallas import tpu as pltpu
from jax.experimental.pallas import tpu_sc as plsc

D = 64
H = 50
B = 4096
BT = 128
NW = 32
NBUF = 4
LAG = 3


@functools.partial(
    pl.kernel,
    mesh=plsc.VectorSubcoreMesh(core_axis_name="c", subcore_axis_name="s"),
    out_type=jax.ShapeDtypeStruct((H, D, B), jnp.float32),
    scratch_types=[
        pltpu.VMEM((H, BT), jnp.int32),
        pltpu.VMEM((NBUF, BT, D), jnp.float32),
        pltpu.VMEM((2, D, BT), jnp.float32),
        pltpu.SemaphoreType.DMA((NBUF,)),
        pltpu.SemaphoreType.DMA((2,)),
        pltpu.SemaphoreType.DMA,
    ],
    compiler_params=pltpu.CompilerParams(use_tc_tiling_on_sc=False, needs_layout_passes=False),
)
def _gather_t(idx_hbm, table_hbm, out_hbm, idx_v, rows_v, y_v, gsem, wsem, isem):
    w = lax.axis_index("s") * 2 + lax.axis_index("c")
    pltpu.async_copy(idx_hbm.at[w], idx_v, isem).wait()
    iota = lax.iota(jnp.int32, 16)

    gd = [None] * H
    wd = [None] * H
    for h in range(LAG):
        gd[h] = pltpu.async_copy(
            table_hbm.at[idx_v.at[h]], rows_v.at[h % NBUF], gsem.at[h % NBUF]
        )
    for h in range(H):
        gd[h].wait()
        if h >= 2:
            wd[h - 2].wait()
        rbuf = rows_v.at[h % NBUF]
        ybuf = y_v.at[h % 2]

        def tbody(d, carry, rbuf=rbuf, ybuf=ybuf):
            col = jnp.broadcast_to(d, (16,))
            for j in range(8):
                vals = plsc.load_gather(rbuf, [iota + (j * 16), col])
                ybuf[d, pl.ds(j * 16, 16)] = vals
            return carry

        lax.fori_loop(0, D, tbody, 0)
        wd[h] = pltpu.async_copy(
            ybuf, out_hbm.at[h, :, pl.ds(w * BT, BT)], wsem.at[h % 2]
        )
        if h + LAG < H:
            nh = h + LAG
            gd[nh] = pltpu.async_copy(
                table_hbm.at[idx_v.at[nh]], rows_v.at[nh % NBUF], gsem.at[nh % NBUF]
            )
    wd[H - 2].wait()
    wd[H - 1].wait()


def kernel(Path, node2vec):
    idx = Path.astype(jnp.int32).reshape(NW, BT, H).transpose(0, 2, 1)
    out = _gather_t(idx, node2vec)
    return jnp.transpose(out, (2, 0, 1))


# speedup vs baseline: 1.4603x; 1.4603x over previous
import functools

import jax
import jax.numpy as jnp
from jax import lax
from jax.experimental import pallas as pl
from jax.experimental.pallas import tpu as pltpu
from jax.experimental.pallas import tpu_sc as plsc

D = 64
H = 50
B = 4096
BT = 128
NW = 32
NBUF = 4
LAG = 3


@functools.partial(
    pl.kernel,
    mesh=plsc.VectorSubcoreMesh(core_axis_name="c", subcore_axis_name="s"),
    out_type=jax.ShapeDtypeStruct((H, D, B), jnp.float32),
    scratch_types=[
        pltpu.VMEM((H, BT), jnp.int32),
        pltpu.VMEM((NBUF, BT, D), jnp.float32),
        pltpu.VMEM((2, D, BT), jnp.float32),
        pltpu.SemaphoreType.DMA((NBUF,)),
        pltpu.SemaphoreType.DMA((2,)),
        pltpu.SemaphoreType.DMA,
    ],
    compiler_params=pltpu.CompilerParams(use_tc_tiling_on_sc=False, needs_layout_passes=False),
)
def _gather_t(idx_hbm, table_hbm, out_hbm, idx_v, rows_v, y_v, gsem, wsem, isem):
    w = lax.axis_index("s") * 2 + lax.axis_index("c")
    pltpu.async_copy(idx_hbm.at[w], idx_v, isem).wait()
    iota = lax.iota(jnp.int32, 16)

    gd = [None] * H
    wd = [None] * H
    for h in range(LAG):
        gd[h] = pltpu.async_copy(
            table_hbm.at[idx_v.at[h]], rows_v.at[h % NBUF], gsem.at[h % NBUF]
        )
    for h in range(H):
        gd[h].wait()
        if h >= 2:
            wd[h - 2].wait()
        rbuf = rows_v.at[h % NBUF]
        ybuf = y_v.at[h % 2]

        @plsc.parallel_loop(0, D, unroll=4)
        def _(d, rbuf=rbuf, ybuf=ybuf):
            col = jnp.broadcast_to(d, (16,))
            for j in range(8):
                vals = plsc.load_gather(rbuf, [iota + (j * 16), col])
                ybuf[d, pl.ds(j * 16, 16)] = vals
        wd[h] = pltpu.async_copy(
            ybuf, out_hbm.at[h, :, pl.ds(w * BT, BT)], wsem.at[h % 2]
        )
        if h + LAG < H:
            nh = h + LAG
            gd[nh] = pltpu.async_copy(
                table_hbm.at[idx_v.at[nh]], rows_v.at[nh % NBUF], gsem.at[nh % NBUF]
            )
    wd[H - 2].wait()
    wd[H - 1].wait()


def kernel(Path, node2vec):
    idx = Path.astype(jnp.int32).reshape(NW, BT, H).transpose(0, 2, 1)
    out = _gather_t(idx, node2vec)
    return jnp.transpose(out, (2, 0, 1))


# speedup vs baseline: 2.4812x; 1.6991x over previous
import functools

import jax
import jax.numpy as jnp
from jax import lax
from jax.experimental import pallas as pl
from jax.experimental.pallas import tpu as pltpu
from jax.experimental.pallas import tpu_sc as plsc

D = 64
H = 50
B = 4096
BT = 128
NW = 32
NBUF = 4
LAG = 3


@functools.partial(
    pl.kernel,
    mesh=plsc.VectorSubcoreMesh(core_axis_name="c", subcore_axis_name="s"),
    out_type=jax.ShapeDtypeStruct((H, D, B), jnp.float32),
    scratch_types=[
        pltpu.VMEM((H, BT), jnp.int32),
        pltpu.VMEM((NBUF, BT, D), jnp.float32),
        pltpu.VMEM((2, D, BT + 1), jnp.float32),
        pltpu.SemaphoreType.DMA((NBUF,)),
        pltpu.SemaphoreType.DMA((2,)),
        pltpu.SemaphoreType.DMA,
    ],
    compiler_params=pltpu.CompilerParams(use_tc_tiling_on_sc=False, needs_layout_passes=False),
)
def _gather_t(idx_hbm, table_hbm, out_hbm, idx_v, rows_v, y_v, gsem, wsem, isem):
    w = lax.axis_index("s") * 2 + lax.axis_index("c")
    pltpu.async_copy(idx_hbm.at[w], idx_v, isem).wait()
    iota = lax.iota(jnp.int32, 16)

    gd = [None] * H
    wd = [None] * H
    for h in range(LAG):
        gd[h] = pltpu.async_copy(
            table_hbm.at[idx_v.at[h]], rows_v.at[h % NBUF], gsem.at[h % NBUF]
        )
    for h in range(H):
        gd[h].wait()
        if h >= 2:
            wd[h - 2].wait()
        rbuf = rows_v.at[h % NBUF]
        ybuf = y_v.at[h % 2]

        @plsc.parallel_loop(0, BT, unroll=8)
        def _(b, rbuf=rbuf, ybuf=ybuf):
            colv = jnp.broadcast_to(b, (16,))
            for k in range(4):
                vals = rbuf[b, pl.ds(k * 16, 16)]
                plsc.store_scatter(ybuf, [iota + (k * 16), colv], vals)
        wd[h] = pltpu.async_copy(
            y_v.at[h % 2, :, pl.ds(0, BT)], out_hbm.at[h, :, pl.ds(w * BT, BT)],
            wsem.at[h % 2]
        )
        if h + LAG < H:
            nh = h + LAG
            gd[nh] = pltpu.async_copy(
                table_hbm.at[idx_v.at[nh]], rows_v.at[nh % NBUF], gsem.at[nh % NBUF]
            )
    wd[H - 2].wait()
    wd[H - 1].wait()


def kernel(Path, node2vec):
    idx = Path.astype(jnp.int32).reshape(NW, BT, H).transpose(0, 2, 1)
    out = _gather_t(idx, node2vec)
    return jnp.transpose(out, (2, 0, 1))


# speedup vs baseline: 3.4890x; 1.4062x over previous
import functools

import jax
import jax.numpy as jnp
from jax import lax
from jax.experimental import pallas as pl
from jax.experimental.pallas import tpu as pltpu
from jax.experimental.pallas import tpu_sc as plsc

D = 64
H = 50
B = 4096
BT = 128
NW = 32
NBUF = 8
LAG = 6


@functools.partial(
    pl.kernel,
    mesh=plsc.VectorSubcoreMesh(core_axis_name="c", subcore_axis_name="s"),
    out_type=jax.ShapeDtypeStruct((H, 8, NW, 8, BT), jnp.float32),
    scratch_types=[
        pltpu.VMEM((H, BT), jnp.int32),
        pltpu.VMEM((NBUF, BT, D), jnp.float32),
        pltpu.VMEM((4, 8, 8, BT + 1), jnp.float32),
        pltpu.SemaphoreType.DMA((NBUF,)),
        pltpu.SemaphoreType.DMA((4,)),
        pltpu.SemaphoreType.DMA,
    ],
    compiler_params=pltpu.CompilerParams(use_tc_tiling_on_sc=False, needs_layout_passes=False),
)
def _gather_t(idx_hbm, table_hbm, out_hbm, idx_v, rows_v, y_v, gsem, wsem, isem):
    w = lax.axis_index("s") * 2 + lax.axis_index("c")
    pltpu.async_copy(idx_hbm.at[:, w], idx_v, isem).wait()
    iota = lax.iota(jnp.int32, 16)

    gd = [None] * H
    wd = [None] * H
    for h in range(LAG):
        gd[h] = pltpu.async_copy(
            table_hbm.at[idx_v.at[h]], rows_v.at[h % NBUF], gsem.at[h % NBUF]
        )
    for h in range(H):
        gd[h].wait()
        if h >= 4:
            wd[h - 4].wait()
        rbuf = rows_v.at[h % NBUF]
        ybuf = y_v.at[h % 4]

        @plsc.parallel_loop(0, BT, unroll=8)
        def _(b, rbuf=rbuf, ybuf=ybuf):
            colv = jnp.broadcast_to(b, (16,))
            for k in range(4):
                vals = rbuf[b, pl.ds(k * 16, 16)]
                plsc.store_scatter(
                    ybuf, [(iota >> 3) + (2 * k), iota & 7, colv], vals
                )
        wd[h] = pltpu.async_copy(
            y_v.at[h % 4, :, :, pl.ds(0, BT)], out_hbm.at[h, :, w],
            wsem.at[h % 4]
        )
        if h + LAG < H:
            nh = h + LAG
            gd[nh] = pltpu.async_copy(
                table_hbm.at[idx_v.at[nh]], rows_v.at[nh % NBUF], gsem.at[nh % NBUF]
            )
    for j in range(H - 4, H):
        wd[j].wait()


def kernel(Path, node2vec):
    idx = jnp.transpose(Path).astype(jnp.int32).reshape(H, NW, BT)
    out = _gather_t(idx, node2vec)
    return jnp.transpose(out, (2, 4, 0, 1, 3)).reshape(B, H, D)
